# Initial kernel scaffold; baseline (speedup 1.0000x reference)
import jax
import jax.numpy as jnp
from jax.experimental import pallas as pl


def kernel(feat, edge_index, edge_type, node_id, graph_ids, rel_labels, relation_triplets, W_proj, b_proj, rel_base, W_rel, W_self, W_msg, W_rel2ent, W_fc, b_fc):
    raise NotImplementedError("write your pallas kernel here")


# speedup vs baseline: 3.8483x; 3.8483x over previous
import functools

import jax
import jax.numpy as jnp
from jax import lax
from jax.experimental import pallas as pl
from jax.experimental.pallas import tpu as pltpu
from jax.experimental.pallas import tpu_sc as plsc

N = 10000
E = 320000
D_IN = 128
D_ENT = 128
D_REL = 32
NREL = 200
B = 64
ERT = 4096

NTILES = 32
CHUNK = 64
NCHUNK = 160
EPT = CHUNK * NCHUNK
EPAD = NTILES * EPT
NROWS = 10112
ROWS_PER_TILE = NROWS // 16
DEGN = 12288
DEG_PER_TILE = DEGN // 16

RBLK = 1000
NBLK = N // RBLK


def _tc1_body(rt_src_ref, rt_dstT_ref, rel_base_ref, W_rel_ref, W_r2e_ref,
              feat_ref, W_proj_ref, b_proj_ref, W_msg_ref, W_self_ref,
              M_ref, S_ref, emb_rel_ref, r2e_ref):
    i = pl.program_id(0)
    e = jnp.maximum(feat_ref[...] @ W_proj_ref[...] + b_proj_ref[...], 0.0)
    M_ref[...] = e @ W_msg_ref[...]
    S_ref[...] = e @ W_self_ref[...]

    @pl.when(i == 0)
    def _():
        rs = rt_src_ref[...]
        rdT = rt_dstT_ref[...]
        oh_s = (jax.lax.broadcasted_iota(jnp.int32, (ERT, NREL), 1) == rs
                ).astype(jnp.float32)
        oh_dT = (jax.lax.broadcasted_iota(jnp.int32, (NREL, ERT), 0) == rdT
                 ).astype(jnp.float32)
        msgs = oh_s @ rel_base_ref[...]
        ragg = oh_dT @ msgs
        deg = jnp.sum(oh_dT, axis=1, keepdims=True)
        er = jnp.maximum((ragg / jnp.maximum(deg, 1.0)) @ W_rel_ref[...]
                         + rel_base_ref[...], 0.0)
        emb_rel_ref[...] = er
        r2e_ref[...] = er @ W_r2e_ref[...]


_tc1_call = pl.pallas_call(
    _tc1_body,
    grid=(NBLK,),
    in_specs=[
        pl.BlockSpec((ERT, 1), lambda i: (0, 0)),
        pl.BlockSpec((1, ERT), lambda i: (0, 0)),
        pl.BlockSpec((NREL, D_REL), lambda i: (0, 0)),
        pl.BlockSpec((D_REL, D_REL), lambda i: (0, 0)),
        pl.BlockSpec((D_REL, D_ENT), lambda i: (0, 0)),
        pl.BlockSpec((RBLK, D_IN), lambda i: (i, 0)),
        pl.BlockSpec((D_IN, D_ENT), lambda i: (0, 0)),
        pl.BlockSpec((1, D_ENT), lambda i: (0, 0)),
        pl.BlockSpec((D_ENT, D_ENT), lambda i: (0, 0)),
        pl.BlockSpec((D_ENT, D_ENT), lambda i: (0, 0)),
    ],
    out_specs=[
        pl.BlockSpec((RBLK, D_ENT), lambda i: (i, 0)),
        pl.BlockSpec((RBLK, D_ENT), lambda i: (i, 0)),
        pl.BlockSpec((NREL, D_REL), lambda i: (0, 0)),
        pl.BlockSpec((NREL, D_ENT), lambda i: (0, 0)),
    ],
    out_shape=[
        jax.ShapeDtypeStruct((N, D_ENT), jnp.float32),
        jax.ShapeDtypeStruct((N, D_ENT), jnp.float32),
        jax.ShapeDtypeStruct((NREL, D_REL), jnp.float32),
        jax.ShapeDtypeStruct((NREL, D_ENT), jnp.float32),
    ],
)


def _sc_body(M_hbm, r2e_hbm, src_hbm, dst_hbm, typ_hbm, zeros_hbm, zdeg_hbm,
             agg_out, deg_out,
             sidx, didx, typv, mrows, r2erows, onesb, agg_s, deg_s, r2e_s,
             isem, gsem, msem, dsem, rsem):
    cid = lax.axis_index("c")
    sid = lax.axis_index("s")

    pltpu.sync_copy(zeros_hbm.at[pl.ds(sid * ROWS_PER_TILE, ROWS_PER_TILE)],
                    agg_s.at[pl.ds(sid * ROWS_PER_TILE, ROWS_PER_TILE)])
    pltpu.sync_copy(zdeg_hbm.at[pl.ds(sid * DEG_PER_TILE, DEG_PER_TILE)],
                    deg_s.at[pl.ds(sid * DEG_PER_TILE, DEG_PER_TILE)])
    @pl.when(sid == 0)
    def _():
        pltpu.sync_copy(r2e_hbm, r2e_s)

    ones16 = jnp.ones((16,), jnp.float32)

    def _fill(j, carry):
        onesb[pl.ds(j * 16, 16)] = ones16
        return carry

    lax.fori_loop(0, CHUNK // 16, _fill, 0)
    plsc.subcore_barrier()

    tile = cid * 16 + sid

    def _chunk(k, carry):
        base = tile * EPT + k * CHUNK
        pltpu.sync_copy(src_hbm.at[pl.ds(base, CHUNK)], sidx.at[0])
        pltpu.sync_copy(dst_hbm.at[pl.ds(base, CHUNK)], didx.at[0])
        pltpu.sync_copy(typ_hbm.at[pl.ds(base, CHUNK)], typv.at[0])
        g = pltpu.async_copy(M_hbm.at[sidx.at[0]], mrows.at[0], gsem.at[0])
        r = pltpu.async_copy(r2e_s.at[typv.at[0]], r2erows, rsem)
        g.wait()
        r.wait()

        def _addr2e(j, carry2):
            for d in range(D_ENT // 16):
                sl = pl.ds(d * 16, 16)
                mrows[0, j, sl] = mrows[0, j, sl] + r2erows[j, sl]
            return carry2

        lax.fori_loop(0, CHUNK, _addr2e, 0)
        pltpu.sync_copy(mrows.at[0], agg_s.at[didx.at[0]], add=True)
        pltpu.sync_copy(onesb, deg_s.at[didx.at[0]], add=True)
        return carry

    lax.fori_loop(0, NCHUNK, _chunk, 0)
    plsc.subcore_barrier()

    pltpu.sync_copy(agg_s.at[pl.ds(sid * ROWS_PER_TILE, ROWS_PER_TILE)],
                    agg_out.at[cid, pl.ds(sid * ROWS_PER_TILE, ROWS_PER_TILE)])
    pltpu.sync_copy(deg_s.at[pl.ds(sid * DEG_PER_TILE, DEG_PER_TILE)],
                    deg_out.at[cid, pl.ds(sid * DEG_PER_TILE, DEG_PER_TILE)])


@functools.lru_cache(maxsize=1)
def _get_sc_call():
  return pl.kernel(
    _sc_body,
    out_type=[
        jax.ShapeDtypeStruct((2, NROWS, D_ENT), jnp.float32),
        jax.ShapeDtypeStruct((2, DEGN), jnp.float32),
    ],
    mesh=plsc.VectorSubcoreMesh(core_axis_name="c", subcore_axis_name="s"),
    scratch_types=[
        pltpu.MemorySpace.VMEM((4, CHUNK), jnp.int32),
        pltpu.MemorySpace.VMEM((4, CHUNK), jnp.int32),
        pltpu.MemorySpace.VMEM((4, CHUNK), jnp.int32),
        pltpu.MemorySpace.VMEM((2, CHUNK, D_ENT), jnp.float32),
        pltpu.MemorySpace.VMEM((CHUNK, D_ENT), jnp.float32),
        pltpu.MemorySpace.VMEM((CHUNK,), jnp.float32),
        pltpu.MemorySpace.VMEM_SHARED((NROWS, D_ENT), jnp.float32),
        pltpu.MemorySpace.VMEM_SHARED((DEGN,), jnp.float32),
        pltpu.MemorySpace.VMEM_SHARED((NREL, D_ENT), jnp.float32),
        pltpu.SemaphoreType.DMA((4,)),
        pltpu.SemaphoreType.DMA((2,)),
        pltpu.SemaphoreType.DMA((2,)),
        pltpu.SemaphoreType.DMA((2,)),
        pltpu.SemaphoreType.DMA,
    ],
  )


def _tc2_body(S_ref, agg_ref, degT_ref, gid_ref, nid_ref, rl_ref, er_ref,
              Wfc_ref, bfc_ref, out_ref, acc_ref):
    i = pl.program_id(0)
    a = agg_ref[0] + agg_ref[1]
    deg = degT_ref[0, 0] + degT_ref[1, 0]
    h = jnp.maximum(S_ref[...] + a / jnp.maximum(deg, 1.0), 0.0)
    w123 = jnp.concatenate([Wfc_ref[0:D_ENT, :],
                            Wfc_ref[D_ENT:2 * D_ENT, :],
                            Wfc_ref[2 * D_ENT:3 * D_ENT, :]], axis=1)
    hw = h @ w123
    nid = nid_ref[0]
    m1 = (nid == 1).astype(jnp.float32)
    m2 = (nid == 2).astype(jnp.float32)
    cols = jnp.concatenate(
        [hw[:, 0:1], hw[:, 1:2] * m1, hw[:, 2:3] * m2,
         jnp.ones((RBLK, 1), jnp.float32)], axis=1)
    gid_row = gid_ref[0]
    ogT = (jax.lax.broadcasted_iota(jnp.int32, (B, RBLK), 0) == gid_row
           ).astype(jnp.float32)
    part = ogT @ cols
    prev = jnp.where(i == 0, jnp.zeros_like(part), acc_ref[:, 0:4])
    total = prev + part
    acc_ref[:, 0:4] = total

    @pl.when(i == NBLK - 1)
    def _():
        w4 = Wfc_ref[3 * D_ENT:3 * D_ENT + D_REL, :]
        erw = er_ref[...] @ w4
        ohr = (jax.lax.broadcasted_iota(jnp.int32, (B, NREL), 1) == rl_ref[...]
               ).astype(jnp.float32)
        relc = ohr @ erw
        out_ref[...] = (total[:, 0:1] / jnp.maximum(total[:, 3:4], 1.0)
                        + total[:, 1:2] + total[:, 2:3] + relc + bfc_ref[...])


_tc2_call = pl.pallas_call(
    _tc2_body,
    grid=(NBLK,),
    in_specs=[
        pl.BlockSpec((RBLK, D_ENT), lambda i: (i, 0)),
        pl.BlockSpec((2, RBLK, D_ENT), lambda i: (0, i, 0)),
        pl.BlockSpec((2, 1, RBLK, 1), lambda i: (0, i, 0, 0)),
        pl.BlockSpec((1, 1, RBLK), lambda i: (i, 0, 0)),
        pl.BlockSpec((1, RBLK, 1), lambda i: (i, 0, 0)),
        pl.BlockSpec((B, 1), lambda i: (0, 0)),
        pl.BlockSpec((NREL, D_REL), lambda i: (0, 0)),
        pl.BlockSpec((3 * D_ENT + D_REL, 1), lambda i: (0, 0)),
        pl.BlockSpec((1, 1), lambda i: (0, 0)),
    ],
    out_specs=pl.BlockSpec((B, 1), lambda i: (0, 0)),
    out_shape=jax.ShapeDtypeStruct((B, 1), jnp.float32),
    scratch_shapes=[pltpu.VMEM((B, 8), jnp.float32)],
)


def kernel(feat, edge_index, edge_type, node_id, graph_ids, rel_labels,
           relation_triplets, W_proj, b_proj, rel_base, W_rel, W_self,
           W_msg, W_rel2ent, W_fc, b_fc):
    rt_src = relation_triplets[:, 0:1]
    rt_dstT = relation_triplets[:, 1].reshape(1, ERT)

    M, S, emb_rel, r2e = _tc1_call(
        rt_src, rt_dstT, rel_base, W_rel, W_rel2ent,
        feat, W_proj, b_proj.reshape(1, D_ENT), W_msg, W_self)

    pad = EPAD - E
    src_p = jnp.concatenate([edge_index[0], jnp.zeros((pad,), jnp.int32)])
    pad_dst = N + (jnp.arange(pad, dtype=jnp.int32) % (NROWS - N))
    dst_p = jnp.concatenate([edge_index[1], pad_dst])
    typ_p = jnp.concatenate([edge_type, jnp.zeros((pad,), jnp.int32)])
    zeros_tbl = jnp.zeros((NROWS, D_ENT), jnp.float32)
    zeros_deg = jnp.zeros((DEGN,), jnp.float32)

    agg_parts, deg_parts = _get_sc_call()(
        M, r2e, src_p, dst_p, typ_p, zeros_tbl, zeros_deg)

    deg4 = deg_parts[:, :N].reshape(2, NBLK, RBLK, 1)
    gid3 = graph_ids.reshape(NBLK, 1, RBLK)
    nid3 = node_id.reshape(NBLK, RBLK, 1)
    rl2 = rel_labels.reshape(B, 1)

    return _tc2_call(S, agg_parts, deg4, gid3, nid3, rl2, emb_rel,
                     W_fc, b_fc.reshape(1, 1))


# speedup vs baseline: 6.4824x; 1.6845x over previous
import functools

import jax
import jax.numpy as jnp
from jax import lax
from jax.experimental import pallas as pl
from jax.experimental.pallas import tpu as pltpu
from jax.experimental.pallas import tpu_sc as plsc

N = 10000
E = 320000
D_IN = 128
D_ENT = 128
D_REL = 32
NREL = 200
B = 64
ERT = 4096

NTILES = 32
CHUNK = 64
NCHUNK = 160
EPT = CHUNK * NCHUNK
EPAD = NTILES * EPT
NROWS = 10112
ROWS_PER_TILE = NROWS // 16
DEGN = 12288
DEG_PER_TILE = DEGN // 16

RBLK = 1000
NBLK = N // RBLK


def _tc1_body(rt_src_ref, rt_dstT_ref, rel_base_ref, W_rel_ref, W_r2e_ref,
              feat_ref, W_proj_ref, b_proj_ref, W_msg_ref, W_self_ref,
              M_ref, S_ref, emb_rel_ref, r2e_ref):
    i = pl.program_id(0)
    e = jnp.maximum(feat_ref[...] @ W_proj_ref[...] + b_proj_ref[...], 0.0)
    M_ref[...] = e @ W_msg_ref[...]
    S_ref[...] = e @ W_self_ref[...]

    @pl.when(i == 0)
    def _():
        rs = rt_src_ref[...]
        rdT = rt_dstT_ref[...]
        oh_s = (jax.lax.broadcasted_iota(jnp.int32, (ERT, NREL), 1) == rs
                ).astype(jnp.float32)
        oh_dT = (jax.lax.broadcasted_iota(jnp.int32, (NREL, ERT), 0) == rdT
                 ).astype(jnp.float32)
        msgs = oh_s @ rel_base_ref[...]
        ragg = oh_dT @ msgs
        deg = jnp.sum(oh_dT, axis=1, keepdims=True)
        er = jnp.maximum((ragg / jnp.maximum(deg, 1.0)) @ W_rel_ref[...]
                         + rel_base_ref[...], 0.0)
        emb_rel_ref[...] = er
        r2e_ref[...] = er @ W_r2e_ref[...]


_tc1_call = pl.pallas_call(
    _tc1_body,
    grid=(NBLK,),
    in_specs=[
        pl.BlockSpec((ERT, 1), lambda i: (0, 0)),
        pl.BlockSpec((1, ERT), lambda i: (0, 0)),
        pl.BlockSpec((NREL, D_REL), lambda i: (0, 0)),
        pl.BlockSpec((D_REL, D_REL), lambda i: (0, 0)),
        pl.BlockSpec((D_REL, D_ENT), lambda i: (0, 0)),
        pl.BlockSpec((RBLK, D_IN), lambda i: (i, 0)),
        pl.BlockSpec((D_IN, D_ENT), lambda i: (0, 0)),
        pl.BlockSpec((1, D_ENT), lambda i: (0, 0)),
        pl.BlockSpec((D_ENT, D_ENT), lambda i: (0, 0)),
        pl.BlockSpec((D_ENT, D_ENT), lambda i: (0, 0)),
    ],
    out_specs=[
        pl.BlockSpec((RBLK, D_ENT), lambda i: (i, 0)),
        pl.BlockSpec((RBLK, D_ENT), lambda i: (i, 0)),
        pl.BlockSpec((NREL, D_REL), lambda i: (0, 0)),
        pl.BlockSpec((NREL, D_ENT), lambda i: (0, 0)),
    ],
    out_shape=[
        jax.ShapeDtypeStruct((N, D_ENT), jnp.float32),
        jax.ShapeDtypeStruct((N, D_ENT), jnp.float32),
        jax.ShapeDtypeStruct((NREL, D_REL), jnp.float32),
        jax.ShapeDtypeStruct((NREL, D_ENT), jnp.float32),
    ],
)


def _sc_body(M_hbm, r2e_hbm, src_hbm, dst_hbm, typ_hbm, zeros_hbm, zdeg_hbm,
             agg_out, deg_out,
             sidx, didx, typv, mrows, r2erows, onesb, agg_s, deg_s, r2e_s,
             isem, gsem, msem, dsem, rsem):
    cid = lax.axis_index("c")
    sid = lax.axis_index("s")

    pltpu.sync_copy(zeros_hbm.at[pl.ds(sid * ROWS_PER_TILE, ROWS_PER_TILE)],
                    agg_s.at[pl.ds(sid * ROWS_PER_TILE, ROWS_PER_TILE)])
    pltpu.sync_copy(zdeg_hbm.at[pl.ds(sid * DEG_PER_TILE, DEG_PER_TILE)],
                    deg_s.at[pl.ds(sid * DEG_PER_TILE, DEG_PER_TILE)])
    @pl.when(sid == 0)
    def _():
        pltpu.sync_copy(r2e_hbm, r2e_s)

    ones16 = jnp.ones((16,), jnp.float32)

    def _fill(j, carry):
        onesb[pl.ds(j * 16, 16)] = ones16
        return carry

    lax.fori_loop(0, CHUNK // 16, _fill, 0)
    plsc.subcore_barrier()

    tile = cid * 16 + sid

    def _fire_idx(cnum, q):
        base = tile * EPT + cnum * CHUNK
        pltpu.async_copy(src_hbm.at[pl.ds(base, CHUNK)], sidx.at[q], isem.at[q])
        pltpu.async_copy(dst_hbm.at[pl.ds(base, CHUNK)], didx.at[q], isem.at[q])
        pltpu.async_copy(typ_hbm.at[pl.ds(base, CHUNK)], typv.at[q], isem.at[q])

    def _wait_idx(q):
        pltpu.make_async_copy(src_hbm.at[pl.ds(0, CHUNK)], sidx.at[q], isem.at[q]).wait()
        pltpu.make_async_copy(dst_hbm.at[pl.ds(0, CHUNK)], didx.at[q], isem.at[q]).wait()
        pltpu.make_async_copy(typ_hbm.at[pl.ds(0, CHUNK)], typv.at[q], isem.at[q]).wait()

    def _fire_mg(q, b):
        pltpu.async_copy(M_hbm.at[sidx.at[q]], mrows.at[b], gsem.at[b])

    def _wait_mg(b):
        pltpu.make_async_copy(M_hbm.at[sidx.at[0]], mrows.at[b], gsem.at[b]).wait()

    def _drain_scatters(b):
        pltpu.make_async_copy(mrows.at[b], agg_s.at[didx.at[0]], msem.at[b]).wait()
        pltpu.make_async_copy(onesb, deg_s.at[didx.at[0]], dsem.at[b]).wait()

    _fire_idx(0, 0)
    _fire_idx(1, 1)
    _wait_idx(0)
    _fire_mg(0, 0)

    def _quad(cc, carry):
        for b4 in range(4):
            c = cc * 4 + b4
            b = b4 % 2
            nb = 1 - b
            q = b4

            @pl.when(c > 0)
            def _():
                _drain_scatters(nb)

            @pl.when(c + 2 < NCHUNK)
            def _():
                _fire_idx(c + 2, (b4 + 2) % 4)

            @pl.when(c + 1 < NCHUNK)
            def _():
                _wait_idx((b4 + 1) % 4)
                _fire_mg((b4 + 1) % 4, nb)

            pltpu.async_copy(r2e_s.at[typv.at[q]], r2erows, rsem)
            _wait_mg(b)
            pltpu.make_async_copy(r2e_s.at[typv.at[0]], r2erows, rsem).wait()

            def _addr2e(j, carry2):
                for d in range(D_ENT // 16):
                    sl = pl.ds(d * 16, 16)
                    mrows[b, j, sl] = mrows[b, j, sl] + r2erows[j, sl]
                return carry2

            lax.fori_loop(0, CHUNK, _addr2e, 0)
            pltpu.async_copy(mrows.at[b], agg_s.at[didx.at[q]], msem.at[b],
                             add=True)
            pltpu.async_copy(onesb, deg_s.at[didx.at[q]], dsem.at[b], add=True)
        return carry

    lax.fori_loop(0, NCHUNK // 4, _quad, 0)

    _drain_scatters(1)
    plsc.subcore_barrier()

    pltpu.sync_copy(agg_s.at[pl.ds(sid * ROWS_PER_TILE, ROWS_PER_TILE)],
                    agg_out.at[cid, pl.ds(sid * ROWS_PER_TILE, ROWS_PER_TILE)])
    pltpu.sync_copy(deg_s.at[pl.ds(sid * DEG_PER_TILE, DEG_PER_TILE)],
                    deg_out.at[cid, pl.ds(sid * DEG_PER_TILE, DEG_PER_TILE)])


@functools.lru_cache(maxsize=1)
def _get_sc_call():
  return pl.kernel(
    _sc_body,
    out_type=[
        jax.ShapeDtypeStruct((2, NROWS, D_ENT), jnp.float32),
        jax.ShapeDtypeStruct((2, DEGN), jnp.float32),
    ],
    mesh=plsc.VectorSubcoreMesh(core_axis_name="c", subcore_axis_name="s"),
    scratch_types=[
        pltpu.MemorySpace.VMEM((4, CHUNK), jnp.int32),
        pltpu.MemorySpace.VMEM((4, CHUNK), jnp.int32),
        pltpu.MemorySpace.VMEM((4, CHUNK), jnp.int32),
        pltpu.MemorySpace.VMEM((2, CHUNK, D_ENT), jnp.float32),
        pltpu.MemorySpace.VMEM((CHUNK, D_ENT), jnp.float32),
        pltpu.MemorySpace.VMEM((CHUNK,), jnp.float32),
        pltpu.MemorySpace.VMEM_SHARED((NROWS, D_ENT), jnp.float32),
        pltpu.MemorySpace.VMEM_SHARED((DEGN,), jnp.float32),
        pltpu.MemorySpace.VMEM_SHARED((NREL, D_ENT), jnp.float32),
        pltpu.SemaphoreType.DMA((4,)),
        pltpu.SemaphoreType.DMA((2,)),
        pltpu.SemaphoreType.DMA((2,)),
        pltpu.SemaphoreType.DMA((2,)),
        pltpu.SemaphoreType.DMA,
    ],
  )


def _tc2_body(S_ref, agg_ref, degT_ref, gid_ref, nid_ref, rl_ref, er_ref,
              Wfc_ref, bfc_ref, out_ref, acc_ref):
    i = pl.program_id(0)
    a = agg_ref[0] + agg_ref[1]
    deg = degT_ref[0, 0] + degT_ref[1, 0]
    h = jnp.maximum(S_ref[...] + a / jnp.maximum(deg, 1.0), 0.0)
    w123 = jnp.concatenate([Wfc_ref[0:D_ENT, :],
                            Wfc_ref[D_ENT:2 * D_ENT, :],
                            Wfc_ref[2 * D_ENT:3 * D_ENT, :]], axis=1)
    hw = h @ w123
    nid = nid_ref[0]
    m1 = (nid == 1).astype(jnp.float32)
    m2 = (nid == 2).astype(jnp.float32)
    cols = jnp.concatenate(
        [hw[:, 0:1], hw[:, 1:2] * m1, hw[:, 2:3] * m2,
         jnp.ones((RBLK, 1), jnp.float32)], axis=1)
    gid_row = gid_ref[0]
    ogT = (jax.lax.broadcasted_iota(jnp.int32, (B, RBLK), 0) == gid_row
           ).astype(jnp.float32)
    part = ogT @ cols
    prev = jnp.where(i == 0, jnp.zeros_like(part), acc_ref[:, 0:4])
    total = prev + part
    acc_ref[:, 0:4] = total

    @pl.when(i == NBLK - 1)
    def _():
        w4 = Wfc_ref[3 * D_ENT:3 * D_ENT + D_REL, :]
        erw = er_ref[...] @ w4
        ohr = (jax.lax.broadcasted_iota(jnp.int32, (B, NREL), 1) == rl_ref[...]
               ).astype(jnp.float32)
        relc = ohr @ erw
        out_ref[...] = (total[:, 0:1] / jnp.maximum(total[:, 3:4], 1.0)
                        + total[:, 1:2] + total[:, 2:3] + relc + bfc_ref[...])


_tc2_call = pl.pallas_call(
    _tc2_body,
    grid=(NBLK,),
    in_specs=[
        pl.BlockSpec((RBLK, D_ENT), lambda i: (i, 0)),
        pl.BlockSpec((2, RBLK, D_ENT), lambda i: (0, i, 0)),
        pl.BlockSpec((2, 1, RBLK, 1), lambda i: (0, i, 0, 0)),
        pl.BlockSpec((1, 1, RBLK), lambda i: (i, 0, 0)),
        pl.BlockSpec((1, RBLK, 1), lambda i: (i, 0, 0)),
        pl.BlockSpec((B, 1), lambda i: (0, 0)),
        pl.BlockSpec((NREL, D_REL), lambda i: (0, 0)),
        pl.BlockSpec((3 * D_ENT + D_REL, 1), lambda i: (0, 0)),
        pl.BlockSpec((1, 1), lambda i: (0, 0)),
    ],
    out_specs=pl.BlockSpec((B, 1), lambda i: (0, 0)),
    out_shape=jax.ShapeDtypeStruct((B, 1), jnp.float32),
    scratch_shapes=[pltpu.VMEM((B, 8), jnp.float32)],
)


def kernel(feat, edge_index, edge_type, node_id, graph_ids, rel_labels,
           relation_triplets, W_proj, b_proj, rel_base, W_rel, W_self,
           W_msg, W_rel2ent, W_fc, b_fc):
    rt_src = relation_triplets[:, 0:1]
    rt_dstT = relation_triplets[:, 1].reshape(1, ERT)

    M, S, emb_rel, r2e = _tc1_call(
        rt_src, rt_dstT, rel_base, W_rel, W_rel2ent,
        feat, W_proj, b_proj.reshape(1, D_ENT), W_msg, W_self)

    pad = EPAD - E
    src_p = jnp.concatenate([edge_index[0], jnp.zeros((pad,), jnp.int32)])
    pad_dst = N + (jnp.arange(pad, dtype=jnp.int32) % (NROWS - N))
    dst_p = jnp.concatenate([edge_index[1], pad_dst])
    typ_p = jnp.concatenate([edge_type, jnp.zeros((pad,), jnp.int32)])
    zeros_tbl = jnp.zeros((NROWS, D_ENT), jnp.float32)
    zeros_deg = jnp.zeros((DEGN,), jnp.float32)

    agg_parts, deg_parts = _get_sc_call()(
        M, r2e, src_p, dst_p, typ_p, zeros_tbl, zeros_deg)

    deg4 = deg_parts[:, :N].reshape(2, NBLK, RBLK, 1)
    gid3 = graph_ids.reshape(NBLK, 1, RBLK)
    nid3 = node_id.reshape(NBLK, RBLK, 1)
    rl2 = rel_labels.reshape(B, 1)

    return _tc2_call(S, agg_parts, deg4, gid3, nid3, rl2, emb_rel,
                     W_fc, b_fc.reshape(1, 1))
